# baseline (speedup 1.0000x reference)
import jax
import jax.numpy as jnp
from jax import lax
from jax.experimental import pallas as pl
from jax.experimental.pallas import tpu as pltpu
from jax.experimental.pallas import tpu_sc as plsc

V = 1000
B, T = 1024, 50
N = B * T
NC, NS, L = 2, 16, 16
NW = NC * NS
NBH = B // 128
NU = T * NBH
UPW = 13
TP = 52
NVC = 6
CW = 168
NT = (21, 21, 21, 21, 21, 20)
VH0 = (0, 21, 42, 63, 84, 105)
IXW = UPW * 128


def _lse_body(table_ref, out_ref):
    t = table_ref[...]
    m = jnp.max(t, axis=1)
    s = jnp.sum(jnp.exp(t - m[:, None]), axis=1)
    out_ref[...] = m + jnp.log(s)


def _row_lse(table):
    return pl.pallas_call(
        _lse_body,
        out_shape=jax.ShapeDtypeStruct((V,), jnp.float32),
    )(table)


def _sc_body(iarr_hbm, tarr_hbm, tbl_hbm, lse_hbm, out_hbm, part_hbm,
             ix_v, tg_v, ip_v, lse_v, g_v, t_v, acc_v,
             gsem_a, gsem_b, ssem_a, ssem_b):
    wid = lax.axis_index("s") * NC + lax.axis_index("c")
    bh = wid % NBH
    tm = wid // NBH
    lanes = lax.iota(jnp.int32, L)

    base = (bh * (TP // UPW) + tm) * IXW
    pltpu.sync_copy(iarr_hbm.at[pl.ds(base, IXW)], ix_v.at[pl.ds(0, IXW)])
    pltpu.sync_copy(tarr_hbm.at[pl.ds(base, IXW)], tg_v.at[pl.ds(0, IXW)])
    pltpu.sync_copy(lse_hbm, lse_v)
    acc_v[...] = jnp.zeros((L,), jnp.float32)

    gsems = (gsem_a, gsem_b)
    ssems = (ssem_a, ssem_b)

    def fill_ip(joff, vc_next, p_next):
        for k in range(0, 128, L):
            iv = ix_v[pl.ds(joff + k, L)]
            iv = jnp.minimum(jnp.maximum(iv, 0), V - 1) + vc_next * V
            ip_v[p_next, pl.ds(k, L)] = iv

    def issue_gather(p):
        pltpu.async_copy(tbl_hbm.at[ip_v.at[p]], g_v.at[p], gsems[p])

    def wait_gather(p):
        pltpu.make_async_copy(tbl_hbm.at[ip_v.at[p]], g_v.at[p],
                              gsems[p]).wait()

    def wait_scatter(p, nt_prev):
        pltpu.make_async_copy(t_v.at[p, pl.ds(0, nt_prev)],
                              out_hbm.at[0, pl.ds(0, nt_prev), 0],
                              ssems[p]).wait()

    fill_ip(0, 0, 0)
    issue_gather(0)

    def unit(j, carry):
        tt = jnp.minimum(tm + 4 * j, T - 1)
        valid = (tm + 4 * j) <= (T - 1)
        joff = j * 128
        for vc in range(NVC):
            p = vc % 2
            wait_gather(p)
            vcn = (vc + 1) % NVC
            jn = joff + (128 if vc == NVC - 1 else 0)
            fill_ip(jn, vcn, 1 - p)
            issue_gather(1 - p)
            nt_prev = NT[vc - 2]
            if vc >= 2:
                wait_scatter(p, nt_prev)
            else:
                @pl.when(j > 0)
                def _():
                    wait_scatter(p, nt_prev)
            nt = NT[vc]

            def tbody(vh, c, p=p):
                for vl in range(8):
                    vcol = jnp.broadcast_to(vh * 8 + vl, (L,))
                    for k in range(0, 128, L):
                        x = plsc.load_gather(g_v.at[p], [lanes + k, vcol])
                        t_v[p, vh, vl, pl.ds(k, L)] = x
                return c

            lax.fori_loop(0, nt, tbody, 0)
            @pl.when(valid)
            def _():
                upd = jnp.zeros((L,), jnp.float32)
                for k in range(0, 128, L):
                    idx16 = ix_v[pl.ds(joff + k, L)]
                    tgt16 = tg_v[pl.ds(joff + k, L)]
                    off16 = tgt16 - vc * CW
                    sel = (off16 >= 0) & (off16 < CW)
                    x16 = plsc.load_gather(g_v.at[p], [lanes + k, off16],
                                           mask=sel)
                    upd = upd - jnp.where(sel, x16, 0.0)
                    if vc == 0:
                        upd = upd + plsc.load_gather(lse_v, [idx16])
                acc_v[...] = acc_v[...] + upd
            pltpu.async_copy(t_v.at[p, pl.ds(0, nt)],
                             out_hbm.at[tt, pl.ds(VH0[vc], nt), bh],
                             ssems[p])
        return carry

    lax.fori_loop(0, UPW, unit, 0)

    wait_scatter(0, NT[4])
    wait_scatter(1, NT[5])
    wait_gather(0)

    pltpu.sync_copy(acc_v, part_hbm.at[wid])


def _sc_gather(iarr, tarr, tbl_r, lse):
    mesh = plsc.VectorSubcoreMesh(core_axis_name="c", subcore_axis_name="s",
                                  num_cores=NC, num_subcores=NS)
    f = pl.kernel(
        _sc_body,
        out_type=(jax.ShapeDtypeStruct((T, V // 8, NBH, 8, 128),
                                       jnp.float32),
                  jax.ShapeDtypeStruct((NW, L), jnp.float32)),
        mesh=mesh,
        scratch_types=[
            pltpu.VMEM((IXW + 128,), jnp.int32),
            pltpu.VMEM((IXW,), jnp.int32),
            pltpu.VMEM((2, 128), jnp.int32),
            pltpu.VMEM((V,), jnp.float32),
            pltpu.VMEM((2, 128, CW), jnp.float32),
            pltpu.VMEM((2, max(NT), 8, 128), jnp.float32),
            pltpu.VMEM((L,), jnp.float32),
            pltpu.SemaphoreType.DMA,
            pltpu.SemaphoreType.DMA,
            pltpu.SemaphoreType.DMA,
            pltpu.SemaphoreType.DMA,
        ],
        compiler_params=pltpu.CompilerParams(needs_layout_passes=False,
                                             use_tc_tiling_on_sc=False),
    )
    return f(iarr, tarr, tbl_r, lse)


def _prep_idx(a):
    at = jnp.pad(a.astype(jnp.int32).T, ((0, TP - T), (0, 0)),
                 mode="edge")
    r = at.reshape(UPW, TP // UPW, NBH, 128)
    return r.transpose(2, 1, 0, 3).reshape(-1)


def kernel(idx, targets, table):
    iarr = _prep_idx(idx)
    tarr = _prep_idx(targets)
    tp = jnp.pad(table, ((0, 0), (0, NVC * CW - V)))
    tbl_r = tp.reshape(V, NVC, CW).transpose(1, 0, 2).reshape(NVC * V, CW)
    lse = _row_lse(table)
    out5, parts = _sc_gather(iarr, tarr, tbl_r, lse)
    logits = out5.transpose(2, 4, 0, 1, 3).reshape(B, T, V)
    loss = parts.sum() / jnp.float32(N)
    return (logits, loss)


# speedup vs baseline: 1.0211x; 1.0211x over previous
import jax
import jax.numpy as jnp
from jax import lax
from jax.experimental import pallas as pl
from jax.experimental.pallas import tpu as pltpu
from jax.experimental.pallas import tpu_sc as plsc

V = 1000
VP = 1024
B, T = 1024, 50
N = B * T
NC, NS, L = 2, 16, 16
NW = NC * NS
BPW = B // NW
CH = T
CHP = 56
NG = BPW // 2
IDXW = BPW * CHP


def _lse_body(table_ref, out_ref):
    t = table_ref[...]
    m = jnp.max(t, axis=1)
    s = jnp.sum(jnp.exp(t - m[:, None]), axis=1)
    out_ref[...] = m + jnp.log(s)


def _row_lse(table):
    return pl.pallas_call(
        _lse_body,
        out_shape=jax.ShapeDtypeStruct((V,), jnp.float32),
    )(table)


def _loss_chunk(rows_v, idx_v, tgt_v, lse_v, off, acc):
    for i in range(0, CH - L + 1, L):
        row_ids = lax.iota(jnp.int32, L) + i
        idx16 = idx_v[pl.ds(off + i, L)]
        tgt16 = tgt_v[pl.ds(off + i, L)]
        lse16 = plsc.load_gather(lse_v, [idx16])
        x16 = plsc.load_gather(rows_v, [row_ids, tgt16])
        acc = acc + (lse16 - x16)
    rem = CH % L
    if rem:
        i = CH - rem
        lanes = lax.iota(jnp.int32, L)
        msk = lanes < rem
        row_ids = jnp.minimum(lanes + i, CH - 1)
        idx16 = idx_v[pl.ds(off + i, L)]
        tgt16 = tgt_v[pl.ds(off + i, L)]
        lse16 = plsc.load_gather(lse_v, [jnp.where(msk, idx16, 0)])
        x16 = plsc.load_gather(rows_v, [row_ids, jnp.where(msk, tgt16, 0)])
        acc = acc + jnp.where(msk, lse16 - x16, 0.0)
    return acc


def _sc_body(idx_hbm, tgt_hbm, table_hbm, lse_hbm, out_hbm, part_hbm,
             idx_v, tgt_v, lse_v, rows_a, rows_b, acc_v,
             gsem_a, gsem_b, ssem_a, ssem_b):
    wid = lax.axis_index("s") * NC + lax.axis_index("c")
    base = wid * IDXW
    bb = wid * BPW
    pltpu.sync_copy(idx_hbm.at[pl.ds(base, IDXW)], idx_v.at[pl.ds(0, IDXW)])
    pltpu.sync_copy(tgt_hbm.at[pl.ds(base, IDXW)], tgt_v.at[pl.ds(0, IDXW)])
    pltpu.sync_copy(lse_hbm, lse_v)

    def gather(c, buf, sem):
        pltpu.async_copy(table_hbm.at[idx_v.at[pl.ds(c * CHP, CH)]], buf, sem)

    def scatter(c, buf, sem):
        pltpu.async_copy(buf, out_hbm.at[bb + c], sem)

    def gather_wait(buf, sem):
        pltpu.make_async_copy(table_hbm.at[idx_v.at[pl.ds(0, CH)]], buf,
                              sem).wait()

    def scatter_wait(buf, sem):
        pltpu.make_async_copy(buf, out_hbm.at[bb], sem).wait()

    gather(0, rows_a, gsem_a)
    gather(1, rows_b, gsem_b)

    def group(g, acc):
        a = 2 * g
        gather_wait(rows_a, gsem_a)
        scatter(a, rows_a, ssem_a)
        acc = _loss_chunk(rows_a, idx_v, tgt_v, lse_v, a * CHP, acc)
        gather_wait(rows_b, gsem_b)
        scatter(a + 1, rows_b, ssem_b)
        acc = _loss_chunk(rows_b, idx_v, tgt_v, lse_v, (a + 1) * CHP, acc)
        scatter_wait(rows_a, ssem_a)
        gather(a + 2, rows_a, gsem_a)
        scatter_wait(rows_b, ssem_b)
        gather(a + 3, rows_b, gsem_b)
        return acc

    acc = lax.fori_loop(0, NG - 1, group, jnp.zeros((L,), jnp.float32))

    a = 2 * (NG - 1)
    gather_wait(rows_a, gsem_a)
    scatter(a, rows_a, ssem_a)
    acc = _loss_chunk(rows_a, idx_v, tgt_v, lse_v, a * CHP, acc)
    gather_wait(rows_b, gsem_b)
    scatter(a + 1, rows_b, ssem_b)
    acc = _loss_chunk(rows_b, idx_v, tgt_v, lse_v, (a + 1) * CHP, acc)
    scatter_wait(rows_a, ssem_a)
    scatter_wait(rows_b, ssem_b)

    acc_v[...] = acc
    pltpu.sync_copy(acc_v, part_hbm.at[wid])


def _sc_gather(idx_p, tgt_p, table, lse):
    mesh = plsc.VectorSubcoreMesh(core_axis_name="c", subcore_axis_name="s",
                                  num_cores=NC, num_subcores=NS)
    f = pl.kernel(
        _sc_body,
        out_type=(jax.ShapeDtypeStruct((B, T, V), jnp.float32),
                  jax.ShapeDtypeStruct((NW, L), jnp.float32)),
        mesh=mesh,
        scratch_types=[
            pltpu.VMEM((IDXW + L,), jnp.int32),
            pltpu.VMEM((IDXW + L,), jnp.int32),
            pltpu.VMEM((V,), jnp.float32),
            pltpu.VMEM((CH, V), jnp.float32),
            pltpu.VMEM((CH, V), jnp.float32),
            pltpu.VMEM((L,), jnp.float32),
            pltpu.SemaphoreType.DMA,
            pltpu.SemaphoreType.DMA,
            pltpu.SemaphoreType.DMA,
            pltpu.SemaphoreType.DMA,
        ],
        compiler_params=pltpu.CompilerParams(needs_layout_passes=False,
                                             use_tc_tiling_on_sc=False),
    )
    return f(idx_p, tgt_p, table, lse)


def kernel(idx, targets, table):
    pad = ((0, 0), (0, CHP - T))
    idx_p = jnp.pad(idx.astype(jnp.int32), pad).reshape(-1)
    tgt_p = jnp.pad(targets.astype(jnp.int32), pad).reshape(-1)
    lse = _row_lse(table)
    logits, parts = _sc_gather(idx_p, tgt_p, table, lse)
    loss = parts.sum() / jnp.float32(N)
    return (logits, loss)
